# Initial kernel scaffold; baseline (speedup 1.0000x reference)
import jax
import jax.numpy as jnp
from jax.experimental import pallas as pl


def kernel(hidden_states, top_k_index, top_k_weights, gate_up_proj, down_proj):
    raise NotImplementedError("write your pallas kernel here")


# speedup vs baseline: 1.1272x; 1.1272x over previous
import functools

import jax
import jax.numpy as jnp
from jax import lax
from jax.experimental import pallas as pl
from jax.experimental.pallas import tpu as pltpu
from jax.experimental.pallas import tpu_sc as plsc

BLOCK = 256
N_WORKERS = 32
GATHER_CHUNK = 32
COMBINE_CHUNK = 16


def _routing_metadata(top_k_index, top_k_weights, E, B):
    T, K = top_k_index.shape
    S = T * K
    S_pad = S + E * B
    NB = S_pad // B
    flat_e = top_k_index.reshape(S)
    oh = (flat_e[:, None] == jnp.arange(E, dtype=jnp.int32)[None, :]).astype(jnp.int32)
    counts = oh.sum(axis=0)
    rank = jnp.sum(jnp.cumsum(oh, axis=0) * oh, axis=1) - 1
    padded_counts = ((counts + B - 1) // B) * B
    pad_end = jnp.cumsum(padded_counts)
    pad_off = pad_end - padded_counts
    pos = jnp.take(pad_off, flat_e) + rank
    total_pad = pad_end[-1]
    src_tok = jnp.zeros((S_pad,), jnp.int32).at[pos].set(
        (jnp.arange(S, dtype=jnp.int32) // K))
    w_pad = jnp.zeros((S_pad,), jnp.float32).at[pos].set(
        top_k_weights.reshape(S).astype(jnp.float32))
    blk_starts = jnp.arange(NB, dtype=jnp.int32) * B
    last_active = jnp.searchsorted(pad_end, total_pad - 1, side="right").astype(jnp.int32)
    block_expert = jnp.minimum(
        jnp.searchsorted(pad_end, blk_starts, side="right").astype(jnp.int32),
        last_active)
    num_active = (total_pad // B).astype(jnp.int32).reshape(1)
    g = pos.reshape(T, K)
    return src_tok, w_pad, block_expert, num_active, g[:, 0], g[:, 1], S_pad, NB


def _mlp_body(be_ref, na_ref, x_ref, gu_ref, dn_ref, w_ref, y_ref):
    inter = dn_ref.shape[2]
    b = pl.program_id(0)

    @pl.when(b < na_ref[0])
    def _():
        x = x_ref[...]
        gu = gu_ref[0]
        h = lax.dot_general(x, gu, (((1,), (1,)), ((), ())),
                            preferred_element_type=jnp.float32)
        gate = h[:, :inter]
        up = h[:, inter:]
        act = gate * jax.nn.sigmoid(gate) * up
        dn = dn_ref[0]
        y = lax.dot_general(act, dn, (((1,), (1,)), ((), ())),
                            preferred_element_type=jnp.float32)
        y_ref[...] = y * w_ref[0, 0, :][:, None]


def _grouped_mlp(x_pad, gate_up_proj, down_proj, w_pad, block_expert, num_active,
                 S_pad, NB, B):
    E, twoI, H = gate_up_proj.shape
    I = twoI // 2
    grid_spec = pltpu.PrefetchScalarGridSpec(
        num_scalar_prefetch=2,
        grid=(NB,),
        in_specs=[
            pl.BlockSpec((B, H), lambda b, be, na: (b, 0)),
            pl.BlockSpec((1, twoI, H), lambda b, be, na: (be[b], 0, 0)),
            pl.BlockSpec((1, H, I), lambda b, be, na: (be[b], 0, 0)),
            pl.BlockSpec((1, 1, B), lambda b, be, na: (b, 0, 0)),
        ],
        out_specs=pl.BlockSpec((B, H), lambda b, be, na: (b, 0)),
    )
    return pl.pallas_call(
        _mlp_body,
        grid_spec=grid_spec,
        out_shape=jax.ShapeDtypeStruct((S_pad, H), jnp.float32),
        compiler_params=pltpu.CompilerParams(
            dimension_semantics=("arbitrary",)),
    )(block_expert, num_active, x_pad, gate_up_proj, down_proj,
      w_pad.reshape(NB, 1, B))


def _make_gather(S_pad, H):
    rows_pw = S_pad // N_WORKERS
    chunk = GATHER_CHUNK
    n_chunks = rows_pw // chunk
    mesh = plsc.VectorSubcoreMesh(core_axis_name="c", subcore_axis_name="s")

    @functools.partial(
        pl.kernel, mesh=mesh,
        out_type=jax.ShapeDtypeStruct((S_pad, H), jnp.float32),
        scratch_types=[
            pltpu.VMEM((chunk,), jnp.int32),
            pltpu.VMEM((chunk, H), jnp.float32),
            pltpu.SemaphoreType.DMA,
        ],
    )
    def gather_k(src_hbm, hid_hbm, out_hbm, idx_v, rows_v, sem):
        wid = lax.axis_index("s") * 2 + lax.axis_index("c")
        base = wid * rows_pw

        def body(i, carry):
            off = base + i * chunk
            pltpu.sync_copy(src_hbm.at[pl.ds(off, chunk)], idx_v)
            pltpu.async_copy(hid_hbm.at[idx_v], rows_v, sem).wait()
            pltpu.sync_copy(rows_v, out_hbm.at[pl.ds(off, chunk)])
            return carry

        lax.fori_loop(0, n_chunks, body, 0)

    return gather_k


def _make_combine(T, H, S_pad):
    toks_pw = T // N_WORKERS
    chunk = COMBINE_CHUNK
    n_chunks = toks_pw // chunk
    mesh = plsc.VectorSubcoreMesh(core_axis_name="c", subcore_axis_name="s")

    @functools.partial(
        pl.kernel, mesh=mesh,
        out_type=jax.ShapeDtypeStruct((T, H), jnp.float32),
        scratch_types=[
            pltpu.VMEM((chunk,), jnp.int32),
            pltpu.VMEM((chunk,), jnp.int32),
            pltpu.VMEM((chunk, H), jnp.float32),
            pltpu.VMEM((chunk, H), jnp.float32),
            pltpu.VMEM((chunk, H), jnp.float32),
            pltpu.SemaphoreType.DMA,
        ],
    )
    def combine_k(g0_hbm, g1_hbm, ypad_hbm, out_hbm, i0_v, i1_v, a_v, b_v, o_v, sem):
        wid = lax.axis_index("s") * 2 + lax.axis_index("c")
        base = wid * toks_pw

        def body(c, carry):
            off = base + c * chunk
            pltpu.sync_copy(g0_hbm.at[pl.ds(off, chunk)], i0_v)
            pltpu.sync_copy(g1_hbm.at[pl.ds(off, chunk)], i1_v)
            cp0 = pltpu.async_copy(ypad_hbm.at[i0_v], a_v, sem)
            cp1 = pltpu.async_copy(ypad_hbm.at[i1_v], b_v, sem)
            cp0.wait()
            cp1.wait()
            for r in range(chunk):
                def col(j, carry2):
                    o_v[r, pl.ds(j * 16, 16)] = (
                        a_v[r, pl.ds(j * 16, 16)] + b_v[r, pl.ds(j * 16, 16)])
                    return carry2
                lax.fori_loop(0, H // 16, col, 0)
            pltpu.sync_copy(o_v, out_hbm.at[pl.ds(off, chunk)])
            return carry

        lax.fori_loop(0, n_chunks, body, 0)

    return combine_k


def kernel(hidden_states, top_k_index, top_k_weights, gate_up_proj, down_proj):
    T, H = hidden_states.shape
    E = gate_up_proj.shape[0]
    B = BLOCK
    (src_tok, w_pad, block_expert, num_active, g0, g1, S_pad, NB) = (
        _routing_metadata(top_k_index, top_k_weights, E, B))

    x_pad = _make_gather(S_pad, H)(src_tok, hidden_states)
    y_pad = _grouped_mlp(x_pad, gate_up_proj, down_proj, w_pad,
                         block_expert, num_active, S_pad, NB, B)
    out = _make_combine(T, H, S_pad)(g0, g1, y_pad)
    return out


# speedup vs baseline: 1.1455x; 1.0162x over previous
import functools

import jax
import jax.numpy as jnp
from jax import lax
from jax.experimental import pallas as pl
from jax.experimental.pallas import tpu as pltpu
from jax.experimental.pallas import tpu_sc as plsc

BLOCK = 256
N_WORKERS = 32
GATHER_CHUNK = 32
COMBINE_CHUNK = 16


def _routing_metadata(top_k_index, top_k_weights, E, B):
    T, K = top_k_index.shape
    S = T * K
    S_pad = S + E * B
    NB = S_pad // B
    flat_e = top_k_index.reshape(S)
    oh = (flat_e[:, None] == jnp.arange(E, dtype=jnp.int32)[None, :]).astype(jnp.int32)
    counts = oh.sum(axis=0)
    rank = jnp.sum(jnp.cumsum(oh, axis=0) * oh, axis=1) - 1
    padded_counts = ((counts + B - 1) // B) * B
    pad_end = jnp.cumsum(padded_counts)
    pad_off = pad_end - padded_counts
    pos = jnp.take(pad_off, flat_e) + rank
    total_pad = pad_end[-1]
    src_tok = jnp.zeros((S_pad,), jnp.int32).at[pos].set(
        (jnp.arange(S, dtype=jnp.int32) // K))
    w_pad = jnp.zeros((S_pad,), jnp.float32).at[pos].set(
        top_k_weights.reshape(S).astype(jnp.float32))
    blk_starts = jnp.arange(NB, dtype=jnp.int32) * B
    last_active = jnp.searchsorted(pad_end, total_pad - 1, side="right").astype(jnp.int32)
    block_expert = jnp.minimum(
        jnp.searchsorted(pad_end, blk_starts, side="right").astype(jnp.int32),
        last_active)
    num_active = (total_pad // B).astype(jnp.int32).reshape(1)
    g = pos.reshape(T, K)
    return src_tok, w_pad, block_expert, num_active, g[:, 0], g[:, 1], S_pad, NB


def _mlp_body(be_ref, na_ref, x_ref, gu_ref, dn_ref, w_ref, y_ref):
    inter = dn_ref.shape[2]
    b = pl.program_id(0)

    @pl.when(b < na_ref[0])
    def _():
        x = x_ref[...]
        gu = gu_ref[0]
        h = lax.dot_general(x, gu, (((1,), (1,)), ((), ())),
                            preferred_element_type=jnp.float32)
        gate = h[:, :inter]
        up = h[:, inter:]
        act = gate * jax.nn.sigmoid(gate) * up
        dn = dn_ref[0]
        y = lax.dot_general(act, dn, (((1,), (1,)), ((), ())),
                            preferred_element_type=jnp.float32)
        y_ref[...] = y * w_ref[0, 0, :][:, None]


def _grouped_mlp(x_pad, gate_up_proj, down_proj, w_pad, block_expert, num_active,
                 S_pad, NB, B):
    E, twoI, H = gate_up_proj.shape
    I = twoI // 2
    grid_spec = pltpu.PrefetchScalarGridSpec(
        num_scalar_prefetch=2,
        grid=(NB,),
        in_specs=[
            pl.BlockSpec((B, H), lambda b, be, na: (b, 0)),
            pl.BlockSpec((1, twoI, H), lambda b, be, na: (be[b], 0, 0)),
            pl.BlockSpec((1, H, I), lambda b, be, na: (be[b], 0, 0)),
            pl.BlockSpec((1, 1, B), lambda b, be, na: (b, 0, 0)),
        ],
        out_specs=pl.BlockSpec((B, H), lambda b, be, na: (b, 0)),
    )
    return pl.pallas_call(
        _mlp_body,
        grid_spec=grid_spec,
        out_shape=jax.ShapeDtypeStruct((S_pad, H), jnp.float32),
        compiler_params=pltpu.CompilerParams(
            dimension_semantics=("arbitrary",)),
    )(block_expert, num_active, x_pad, gate_up_proj, down_proj,
      w_pad.reshape(NB, 1, B))


def _make_gather(S_pad, H):
    rows_pw = S_pad // N_WORKERS
    chunk = GATHER_CHUNK
    n_chunks = rows_pw // chunk
    mesh = plsc.VectorSubcoreMesh(core_axis_name="c", subcore_axis_name="s")

    @functools.partial(
        pl.kernel, mesh=mesh,
        out_type=jax.ShapeDtypeStruct((S_pad, H), jnp.float32),
        scratch_types=[
            pltpu.VMEM((chunk,), jnp.int32),
            pltpu.VMEM((chunk, H), jnp.float32),
            pltpu.SemaphoreType.DMA,
        ],
    )
    def gather_k(src_hbm, hid_hbm, out_hbm, idx_v, rows_v, sem):
        wid = lax.axis_index("s") * 2 + lax.axis_index("c")
        base = wid * rows_pw

        def body(i, carry):
            off = base + i * chunk
            pltpu.sync_copy(src_hbm.at[pl.ds(off, chunk)], idx_v)
            pltpu.async_copy(hid_hbm.at[idx_v], rows_v, sem).wait()
            pltpu.sync_copy(rows_v, out_hbm.at[pl.ds(off, chunk)])
            return carry

        lax.fori_loop(0, n_chunks, body, 0)

    return gather_k


def _make_pair_gather(T, H, S_pad):
    toks_pw = T // N_WORKERS
    chunk = GATHER_CHUNK
    n_chunks = toks_pw // chunk
    mesh = plsc.VectorSubcoreMesh(core_axis_name="c", subcore_axis_name="s")

    @functools.partial(
        pl.kernel, mesh=mesh,
        out_type=(jax.ShapeDtypeStruct((T, H), jnp.float32),
                  jax.ShapeDtypeStruct((T, H), jnp.float32)),
        scratch_types=[
            pltpu.VMEM((chunk,), jnp.int32),
            pltpu.VMEM((chunk, H), jnp.float32),
            pltpu.SemaphoreType.DMA,
        ],
    )
    def pair_gather_k(g0_hbm, g1_hbm, ypad_hbm, y0_hbm, y1_hbm, idx_v, rows_v, sem):
        wid = lax.axis_index("s") * 2 + lax.axis_index("c")
        base = wid * toks_pw

        def body(c, carry):
            off = base + c * chunk
            pltpu.sync_copy(g0_hbm.at[pl.ds(off, chunk)], idx_v)
            pltpu.async_copy(ypad_hbm.at[idx_v], rows_v, sem).wait()
            pltpu.sync_copy(rows_v, y0_hbm.at[pl.ds(off, chunk)])
            pltpu.sync_copy(g1_hbm.at[pl.ds(off, chunk)], idx_v)
            pltpu.async_copy(ypad_hbm.at[idx_v], rows_v, sem).wait()
            pltpu.sync_copy(rows_v, y1_hbm.at[pl.ds(off, chunk)])
            return carry

        lax.fori_loop(0, n_chunks, body, 0)

    return pair_gather_k


def _add_body(a_ref, b_ref, o_ref):
    o_ref[...] = a_ref[...] + b_ref[...]


def _tc_add(y0, y1, T, H):
    BT = 512
    return pl.pallas_call(
        _add_body,
        grid=(T // BT,),
        in_specs=[pl.BlockSpec((BT, H), lambda i: (i, 0)),
                  pl.BlockSpec((BT, H), lambda i: (i, 0))],
        out_specs=pl.BlockSpec((BT, H), lambda i: (i, 0)),
        out_shape=jax.ShapeDtypeStruct((T, H), jnp.float32),
    )(y0, y1)


def kernel(hidden_states, top_k_index, top_k_weights, gate_up_proj, down_proj):
    T, H = hidden_states.shape
    E = gate_up_proj.shape[0]
    B = BLOCK
    (src_tok, w_pad, block_expert, num_active, g0, g1, S_pad, NB) = (
        _routing_metadata(top_k_index, top_k_weights, E, B))

    x_pad = _make_gather(S_pad, H)(src_tok, hidden_states)
    y_pad = _grouped_mlp(x_pad, gate_up_proj, down_proj, w_pad,
                         block_expert, num_active, S_pad, NB, B)
    y0, y1 = _make_pair_gather(T, H, S_pad)(g0, g1, y_pad)
    return _tc_add(y0, y1, T, H)


# speedup vs baseline: 2.6171x; 2.2848x over previous
import functools

import jax
import jax.numpy as jnp
from jax import lax
from jax.experimental import pallas as pl
from jax.experimental.pallas import tpu as pltpu
from jax.experimental.pallas import tpu_sc as plsc

BLOCK = 256
N_WORKERS = 32
GATHER_CHUNK = 32
COMBINE_CHUNK = 16


def _routing_metadata(top_k_index, top_k_weights, E, B):
    T, K = top_k_index.shape
    S = T * K
    S_pad = S + E * B
    NB = S_pad // B
    flat_e = top_k_index.reshape(S)
    oh = (flat_e[:, None] == jnp.arange(E, dtype=jnp.int32)[None, :]).astype(jnp.int32)
    counts = oh.sum(axis=0)
    rank = jnp.sum(jnp.cumsum(oh, axis=0) * oh, axis=1) - 1
    padded_counts = ((counts + B - 1) // B) * B
    pad_end = jnp.cumsum(padded_counts)
    pad_off = pad_end - padded_counts
    pos = (jnp.take(pad_off, flat_e) + rank).astype(jnp.int32)
    total_pad = pad_end[-1]
    blk_starts = jnp.arange(NB, dtype=jnp.int32) * B
    last_active = jnp.sum((pad_end <= total_pad - 1).astype(jnp.int32))
    block_expert = jnp.minimum(
        jnp.sum((pad_end[None, :] <= blk_starts[:, None]).astype(jnp.int32), axis=1),
        last_active).astype(jnp.int32)
    num_active = (total_pad // B).astype(jnp.int32).reshape(1)
    g = pos.reshape(T, K)
    return pos, block_expert, num_active, g[:, 0], g[:, 1], S_pad, NB


def _mlp_body(be_ref, na_ref, x_ref, gu_ref, dn_ref, y_ref):
    inter = dn_ref.shape[2]
    b = pl.program_id(0)

    @pl.when(b < na_ref[0])
    def _():
        x = x_ref[...]
        gu = gu_ref[0]
        h = lax.dot_general(x, gu, (((1,), (1,)), ((), ())),
                            preferred_element_type=jnp.float32)
        gate = h[:, :inter]
        up = h[:, inter:]
        act = gate * jax.nn.sigmoid(gate) * up
        dn = dn_ref[0]
        y_ref[...] = lax.dot_general(act, dn, (((1,), (1,)), ((), ())),
                                     preferred_element_type=jnp.float32)


def _grouped_mlp(x_pad, gate_up_proj, down_proj, block_expert, num_active,
                 S_pad, NB, B):
    E, twoI, H = gate_up_proj.shape
    I = twoI // 2
    grid_spec = pltpu.PrefetchScalarGridSpec(
        num_scalar_prefetch=2,
        grid=(NB,),
        in_specs=[
            pl.BlockSpec((B, H), lambda b, be, na: (b, 0)),
            pl.BlockSpec((1, twoI, H), lambda b, be, na: (be[b], 0, 0)),
            pl.BlockSpec((1, H, I), lambda b, be, na: (be[b], 0, 0)),
        ],
        out_specs=pl.BlockSpec((B, H), lambda b, be, na: (b, 0)),
    )
    return pl.pallas_call(
        _mlp_body,
        grid_spec=grid_spec,
        out_shape=jax.ShapeDtypeStruct((S_pad, H), jnp.float32),
        compiler_params=pltpu.CompilerParams(
            dimension_semantics=("arbitrary",)),
    )(block_expert, num_active, x_pad, gate_up_proj, down_proj)


def _make_dispatch(S, S_pad, H):
    pairs_pw = S // N_WORKERS
    chunk = GATHER_CHUNK
    n_chunks = pairs_pw // chunk
    mesh = plsc.VectorSubcoreMesh(core_axis_name="c", subcore_axis_name="s")

    @functools.partial(
        pl.kernel, mesh=mesh,
        out_type=jax.ShapeDtypeStruct((S_pad, H), jnp.float32),
        scratch_types=[
            pltpu.VMEM((chunk,), jnp.int32),
            pltpu.VMEM((chunk,), jnp.int32),
            pltpu.VMEM((chunk, H), jnp.float32),
            pltpu.SemaphoreType.DMA,
        ],
    )
    def dispatch_k(tok_hbm, pos_hbm, hid_hbm, out_hbm, it_v, ip_v, rows_v, sem):
        wid = lax.axis_index("s") * 2 + lax.axis_index("c")
        base = wid * pairs_pw

        def body(i, carry):
            off = base + i * chunk
            pltpu.sync_copy(tok_hbm.at[pl.ds(off, chunk)], it_v)
            pltpu.sync_copy(pos_hbm.at[pl.ds(off, chunk)], ip_v)
            pltpu.async_copy(hid_hbm.at[it_v], rows_v, sem).wait()
            pltpu.async_copy(rows_v, out_hbm.at[ip_v], sem).wait()
            return carry

        lax.fori_loop(0, n_chunks, body, 0)

    return dispatch_k


def _make_pair_gather(T, H, S_pad):
    toks_pw = T // N_WORKERS
    chunk = GATHER_CHUNK
    n_chunks = toks_pw // chunk
    mesh = plsc.VectorSubcoreMesh(core_axis_name="c", subcore_axis_name="s")

    @functools.partial(
        pl.kernel, mesh=mesh,
        out_type=(jax.ShapeDtypeStruct((T, H), jnp.float32),
                  jax.ShapeDtypeStruct((T, H), jnp.float32)),
        scratch_types=[
            pltpu.VMEM((chunk,), jnp.int32),
            pltpu.VMEM((chunk, H), jnp.float32),
            pltpu.SemaphoreType.DMA,
        ],
    )
    def pair_gather_k(g0_hbm, g1_hbm, ypad_hbm, y0_hbm, y1_hbm, idx_v, rows_v, sem):
        wid = lax.axis_index("s") * 2 + lax.axis_index("c")
        base = wid * toks_pw

        def body(c, carry):
            off = base + c * chunk
            pltpu.sync_copy(g0_hbm.at[pl.ds(off, chunk)], idx_v)
            pltpu.async_copy(ypad_hbm.at[idx_v], rows_v, sem).wait()
            pltpu.sync_copy(rows_v, y0_hbm.at[pl.ds(off, chunk)])
            pltpu.sync_copy(g1_hbm.at[pl.ds(off, chunk)], idx_v)
            pltpu.async_copy(ypad_hbm.at[idx_v], rows_v, sem).wait()
            pltpu.sync_copy(rows_v, y1_hbm.at[pl.ds(off, chunk)])
            return carry

        lax.fori_loop(0, n_chunks, body, 0)

    return pair_gather_k


def _combine_body(a_ref, b_ref, w0_ref, w1_ref, o_ref):
    o_ref[...] = (a_ref[...] * w0_ref[0, 0, :][:, None]
                  + b_ref[...] * w1_ref[0, 0, :][:, None])


def _tc_combine(y0, y1, top_k_weights, T, H):
    BT = 512
    w0 = top_k_weights[:, 0].reshape(T // BT, 1, BT)
    w1 = top_k_weights[:, 1].reshape(T // BT, 1, BT)
    return pl.pallas_call(
        _combine_body,
        grid=(T // BT,),
        in_specs=[pl.BlockSpec((BT, H), lambda i: (i, 0)),
                  pl.BlockSpec((BT, H), lambda i: (i, 0)),
                  pl.BlockSpec((1, 1, BT), lambda i: (i, 0, 0)),
                  pl.BlockSpec((1, 1, BT), lambda i: (i, 0, 0))],
        out_specs=pl.BlockSpec((BT, H), lambda i: (i, 0)),
        out_shape=jax.ShapeDtypeStruct((T, H), jnp.float32),
    )(y0, y1, w0, w1)


def kernel(hidden_states, top_k_index, top_k_weights, gate_up_proj, down_proj):
    T, H = hidden_states.shape
    K = top_k_index.shape[1]
    E = gate_up_proj.shape[0]
    B = BLOCK
    S = T * K
    (pos, block_expert, num_active, g0, g1, S_pad, NB) = (
        _routing_metadata(top_k_index, top_k_weights, E, B))
    tok = jnp.arange(S, dtype=jnp.int32) // K

    x_pad = _make_dispatch(S, S_pad, H)(tok, pos, hidden_states)
    y_pad = _grouped_mlp(x_pad, gate_up_proj, down_proj,
                         block_expert, num_active, S_pad, NB, B)
    y0, y1 = _make_pair_gather(T, H, S_pad)(g0, g1, y_pad)
    return _tc_combine(y0, y1, top_k_weights, T, H)
